# baseline (speedup 1.0000x reference)
import functools

import jax
import jax.numpy as jnp
from jax import lax
from jax.experimental import pallas as pl
from jax.experimental.pallas import tpu as pltpu
from jax.experimental.pallas import tpu_sc as plsc

_TC_BLOCK_ROWS = 2048
_SC_CHUNK_ROWS = 32


def _tc_copy_body(pe_ref, out_ref):
    out_ref[...] = pe_ref[...]


def _make_sc_copy(row_start, rows, hidden, dtype):
    info = plsc.get_sparse_core_info()
    nc, ns = info.num_cores, info.num_subcores
    nw = nc * ns
    rows_per_w = rows // nw
    n_chunks = rows_per_w // _SC_CHUNK_ROWS
    mesh = plsc.VectorSubcoreMesh(core_axis_name="c", subcore_axis_name="s")

    @functools.partial(
        pl.kernel,
        mesh=mesh,
        out_type=jax.ShapeDtypeStruct((rows, hidden), dtype),
        scratch_types=[
            pltpu.VMEM((2, _SC_CHUNK_ROWS, hidden), dtype),
            pltpu.SemaphoreType.DMA,
            pltpu.SemaphoreType.DMA,
            pltpu.SemaphoreType.DMA,
            pltpu.SemaphoreType.DMA,
        ],
    )
    def sc_copy(pe_hbm, out_hbm, buf, isem0, isem1, osem0, osem1):
        wid = lax.axis_index("s") * nc + lax.axis_index("c")
        base = wid * rows_per_w
        isem = (isem0, isem1)
        osem = (osem0, osem1)

        def src(c):
            return pe_hbm.at[pl.ds(row_start + base + c * _SC_CHUNK_ROWS, _SC_CHUNK_ROWS)]

        def dst(c):
            return out_hbm.at[pl.ds(base + c * _SC_CHUNK_ROWS, _SC_CHUNK_ROWS)]

        ind = {0: pltpu.async_copy(src(0), buf.at[0], isem[0])}
        outd = {}
        for c in range(n_chunks):
            b = c % 2
            ind[c].wait()
            outd[c] = pltpu.async_copy(buf.at[b], dst(c), osem[b])
            if c + 1 < n_chunks:
                if c - 1 >= 0:
                    outd[c - 1].wait()
                ind[c + 1] = pltpu.async_copy(src(c + 1), buf.at[1 - b], isem[1 - b])
        if n_chunks >= 2:
            outd[n_chunks - 2].wait()
        outd[n_chunks - 1].wait()

    return sc_copy


def kernel(x, pe):
    seq_len = x.shape[1]
    hidden = pe.shape[1]
    tc_rows = 6144
    sc_rows = seq_len - tc_rows
    tc_out = pl.pallas_call(
        _tc_copy_body,
        grid=(tc_rows // _TC_BLOCK_ROWS,),
        in_specs=[pl.BlockSpec((_TC_BLOCK_ROWS, hidden), lambda i: (i, 0))],
        out_specs=pl.BlockSpec((_TC_BLOCK_ROWS, hidden), lambda i: (i, 0)),
        out_shape=jax.ShapeDtypeStruct((tc_rows, hidden), pe.dtype),
    )(pe)
    sc_out = _make_sc_copy(tc_rows, sc_rows, hidden, pe.dtype)(pe)
    out = jnp.concatenate([tc_out, sc_out], axis=0)
    return out[None, ...]


# speedup vs baseline: 2.7272x; 2.7272x over previous
import jax
import jax.numpy as jnp
from jax.experimental import pallas as pl
from jax.experimental.pallas import tpu as pltpu

_CHUNKS = (128, 128, 256, 512, 1024, 2048, 4096)


def _copy_body(pe_ref, out_ref, buf, *sems):
    n = len(_CHUNKS)
    isems = sems[:n]
    osems = sems[n:]
    offs = []
    o = 0
    for c in _CHUNKS:
        offs.append(o)
        o += c
    ins = []
    for i, (o, c) in enumerate(zip(offs, _CHUNKS)):
        cp = pltpu.make_async_copy(
            pe_ref.at[pl.ds(o, c)], buf.at[pl.ds(o, c)], isems[i]
        )
        cp.start()
        ins.append(cp)
    outs = []
    for i, (o, c) in enumerate(zip(offs, _CHUNKS)):
        ins[i].wait()
        cp = pltpu.make_async_copy(
            buf.at[pl.ds(o, c)], out_ref.at[pl.ds(o, c)], osems[i]
        )
        cp.start()
        outs.append(cp)
    for cp in outs:
        cp.wait()


def kernel(x, pe):
    seq_len = x.shape[1]
    hidden = pe.shape[1]
    out = pl.pallas_call(
        _copy_body,
        in_specs=[pl.BlockSpec(memory_space=pl.ANY)],
        out_specs=pl.BlockSpec(memory_space=pl.ANY),
        out_shape=jax.ShapeDtypeStruct((seq_len, hidden), pe.dtype),
        scratch_shapes=(
            [pltpu.VMEM((seq_len, hidden), pe.dtype)]
            + [pltpu.SemaphoreType.DMA] * (2 * len(_CHUNKS))
        ),
    )(pe)
    return out[None, ...]
